# baseline (speedup 1.0000x reference)
import jax
import jax.numpy as jnp
from jax.experimental import pallas as pl
from jax.experimental.pallas import tpu as pltpu

_J = 21
_H = 64
_IN = 3
_F32 = jnp.float32


def _rgnn_kernel(enc_ref, dec064_ref, dec0r_ref,
                 w64_ref, we1_ref, be1_ref, we2_ref, be2_ref,
                 wn1_ref, bn1_ref, wi_ref, bi_ref, wh_ref, bh_ref,
                 wout_ref, bout_ref, wdec_ref, bdec_ref, win_ref,
                 out_ref):
    Bb = dec064_ref.shape[0]
    J, H = _J, _H
    R = Bb * J

    w64 = w64_ref[...]
    we1 = we1_ref[...]
    w_e1a, w_e1b = we1[:H], we1[H:]
    b_e1 = be1_ref[...]
    w_e2 = we2_ref[...]
    b_e2 = be2_ref[...]
    wn1 = wn1_ref[...]
    w_n1a, w_n1b = wn1[:H], wn1[H:]
    b_n1 = bn1_ref[...]
    w_i = wi_ref[...]
    b_i = bi_ref[...]
    w_h = wh_ref[...]
    b_h = bh_ref[...]
    w_in = win_ref[...]
    w_od = jnp.dot(wout_ref[...], wdec_ref[...], preferred_element_type=_F32)
    b_od = jnp.dot(bout_ref[...], wdec_ref[...], preferred_element_type=_F32) \
        + bdec_ref[...]

    bf = jnp.bfloat16
    hc = lambda v: v.astype(bf)
    w64h = hc(w64)
    w_abn = hc(jnp.concatenate([w_e1a, w_e1b, w_n1a], axis=1))
    w_en = hc(jnp.dot(jnp.concatenate([w_e2, w_e2], axis=0) * (1.0 / (J - 1)),
                      w_n1b, preferred_element_type=_F32))
    b_n1e = b_n1 + jnp.dot(b_e2, w_n1b, preferred_element_type=_F32)
    b_rz = b_i[:, :2 * H] + b_h[:, :2 * H]
    b_ig, b_hg = b_i[:, 2 * H:], b_h[:, 2 * H:]
    w_ih, w_hh, w_odh = hc(w_i), hc(w_h), hc(w_od)

    def embed_pre(x64):
        pw = jnp.dot(hc(x64), w64h, preferred_element_type=_F32)
        return jnp.concatenate(
            [pw[:, H * j:H * (j + 1)] for j in range(J)], axis=0)

    def embed_nf(x64):
        pw = hc(jnp.dot(hc(x64), w64h, preferred_element_type=_F32))
        return jnp.maximum(jnp.concatenate(
            [pw[:, H * j:H * (j + 1)] for j in range(J)], axis=0), bf(0.0))

    def step(nf, h):
        abn = jnp.dot(nf, w_abn, preferred_element_type=_F32)
        a3 = hc(abn[:, :H]).reshape(J, Bb, H)
        bb3 = hc(abn[:, H:2 * H] + b_e1).reshape(J, Bb, H)
        a2s = [jnp.concatenate([a3[i:i + 1], a3[i + 1:i + 2]], axis=-1)
               for i in range(0, J - 1, 2)]
        a20 = a3[J - 1:J]
        chunks = []
        for j in range(J):
            bb_j = bb3[j:j + 1]
            bb2_j = jnp.concatenate([bb_j, bb_j], axis=-1)
            acc_j = jnp.concatenate(
                [jnp.maximum(a20 + bb_j, bf(0.0)),
                 -jnp.maximum(a3[j:j + 1] + bb_j, bf(0.0))], axis=-1)
            for a2 in a2s:
                acc_j = acc_j + jnp.maximum(a2 + bb2_j, bf(0.0))
            chunks.append(acc_j)
        acc2 = jnp.concatenate(chunks, axis=0)
        nf2 = hc(jnp.maximum(
            abn[:, 2 * H:]
            + jnp.dot(acc2.reshape(R, 2 * H), w_en,
                      preferred_element_type=_F32)
            + b_n1e, 0.0))
        gi = jnp.dot(nf2, w_ih, preferred_element_type=_F32)
        gh = jnp.dot(hc(h), w_hh, preferred_element_type=_F32)
        rz = jax.nn.sigmoid(gi[:, :2 * H] + gh[:, :2 * H] + b_rz)
        r, z = rz[:, :H], rz[:, H:]
        g = jnp.tanh(gi[:, 2 * H:] + b_ig + r * (gh[:, 2 * H:] + b_hg))
        return g + z * (h - g)

    def enc_body(t, h):
        x64 = enc_ref[pl.ds(t, 1)].reshape(Bb, H)
        return step(embed_nf(x64), h)

    h = jax.lax.fori_loop(0, enc_ref.shape[0], enc_body,
                          jnp.zeros((R, H), _F32), unroll=2)

    pre0 = embed_pre(dec064_ref[...])
    pred0 = dec0r_ref[...].reshape(R, _IN)

    def dec_body(t, carry):
        pre, pred, h = carry
        h2 = step(hc(jnp.maximum(pre, 0.0)), h)
        delta = jnp.dot(h2.astype(bf), w_odh,
                        preferred_element_type=_F32) + b_od
        pred2 = pred + delta
        pred63 = jnp.concatenate(
            [pred2[j * Bb:(j + 1) * Bb] for j in range(J)], axis=1)
        out_ref[pl.ds(t, 1)] = pred63.reshape(1, Bb, J * _IN)
        pre2 = pre + jnp.dot(delta, w_in, preferred_element_type=_F32)
        return (pre2, pred2, h2)

    jax.lax.fori_loop(0, out_ref.shape[0], dec_body, (pre0, pred0, h),
                      unroll=2)


def kernel(encoder_input, decoder_input, W_in, b_in, W_e1, b_e1, W_e2, b_e2,
           W_n1, b_n1, W_i, b_i, W_h, b_h, W_out, b_out, W_dec, b_dec,
           send_idx, rec_idx):
    del send_idx, rec_idx
    T_src, B = encoder_input.shape[0], encoder_input.shape[1]
    T_tgt = decoder_input.shape[0]
    J, H = _J, _H
    F = J * _IN

    ones_col = lambda x: jnp.concatenate(
        [x, jnp.ones(x.shape[:-1] + (1,), _F32)], axis=-1)
    enc = ones_col(encoder_input.reshape(T_src, B, F))
    dec064 = ones_col(decoder_input[0].reshape(B, F))
    dec0r = decoder_input[0].transpose(1, 0, 2)
    W64 = jnp.concatenate(
        [jnp.kron(jnp.eye(J, dtype=_F32), W_in),
         jnp.tile(b_in, J).reshape(1, J * H)], axis=0)
    row = lambda v: v.reshape(1, -1)

    Bb = 256
    grid = (B // Bb,)

    wspec = lambda a: pl.BlockSpec(a.shape, lambda i: (0,) * a.ndim)
    weights = (W64, W_e1, row(b_e1), W_e2, row(b_e2),
               W_n1, row(b_n1), W_i, row(b_i), W_h, row(b_h),
               W_out, row(b_out), W_dec, row(b_dec), W_in)

    out = pl.pallas_call(
        _rgnn_kernel,
        grid=grid,
        in_specs=[
            pl.BlockSpec((T_src, Bb, H), lambda i: (0, i, 0)),
            pl.BlockSpec((Bb, H), lambda i: (i, 0)),
            pl.BlockSpec((J, Bb, _IN), lambda i: (0, i, 0)),
        ] + [wspec(w) for w in weights],
        out_specs=pl.BlockSpec((T_tgt, Bb, F), lambda i: (0, i, 0)),
        out_shape=jax.ShapeDtypeStruct((T_tgt, B, F), _F32),
        compiler_params=pltpu.CompilerParams(
            dimension_semantics=("parallel",)),
    )(enc, dec064, dec0r, *weights)
    return out


# speedup vs baseline: 1.0102x; 1.0102x over previous
import jax
import jax.numpy as jnp
from jax.experimental import pallas as pl
from jax.experimental.pallas import tpu as pltpu

_J = 21
_H = 64
_IN = 3
_F32 = jnp.float32


def _rgnn_kernel(enc_ref, dec064_ref, dec0r_ref,
                 w64_ref, we1_ref, be1_ref, we2_ref, be2_ref,
                 wn1_ref, bn1_ref, wi_ref, bi_ref, wh_ref, bh_ref,
                 wout_ref, bout_ref, wdec_ref, bdec_ref, win_ref,
                 out_ref):
    Bb = dec064_ref.shape[0]
    J, H = _J, _H
    R = Bb * J

    w64 = w64_ref[...]
    we1 = we1_ref[...]
    w_e1a, w_e1b = we1[:H], we1[H:]
    b_e1 = be1_ref[...]
    w_e2 = we2_ref[...]
    b_e2 = be2_ref[...]
    wn1 = wn1_ref[...]
    w_n1a, w_n1b = wn1[:H], wn1[H:]
    b_n1 = bn1_ref[...]
    w_i = wi_ref[...]
    b_i = bi_ref[...]
    w_h = wh_ref[...]
    b_h = bh_ref[...]
    w_in = win_ref[...]
    w_od = jnp.dot(wout_ref[...], wdec_ref[...], preferred_element_type=_F32)
    b_od = jnp.dot(bout_ref[...], wdec_ref[...], preferred_element_type=_F32) \
        + bdec_ref[...]

    bf = jnp.bfloat16
    hc = lambda v: v.astype(bf)
    w64h = hc(w64)
    w_abn = hc(jnp.concatenate([w_e1a, w_e1b, w_n1a], axis=1))
    w_en = hc(jnp.dot(jnp.concatenate([w_e2, w_e2], axis=0) * (1.0 / (J - 1)),
                      w_n1b, preferred_element_type=_F32))
    b_n1e = b_n1 + jnp.dot(b_e2, w_n1b, preferred_element_type=_F32)
    b_rz = b_i[:, :2 * H] + b_h[:, :2 * H]
    b_ig, b_hg = b_i[:, 2 * H:], b_h[:, 2 * H:]
    w_ih, w_hh, w_odh = hc(w_i), hc(w_h), hc(w_od)

    def embed_pre(x64):
        pw = jnp.dot(hc(x64), w64h, preferred_element_type=_F32)
        return jnp.concatenate(
            [pw[:, H * j:H * (j + 1)] for j in range(J)], axis=0)

    def embed_nf(x64):
        pw = hc(jnp.dot(hc(x64), w64h, preferred_element_type=_F32))
        return jnp.maximum(jnp.concatenate(
            [pw[:, H * j:H * (j + 1)] for j in range(J)], axis=0), bf(0.0))

    def step(nf, h):
        abn = jnp.dot(nf, w_abn, preferred_element_type=_F32)
        a3 = hc(abn[:, :H]).reshape(J, Bb, H)
        bb3 = hc(abn[:, H:2 * H] + b_e1).reshape(J, Bb, H)
        bb2 = jnp.concatenate([bb3, bb3], axis=-1)
        acc2 = jnp.concatenate([jnp.maximum(a3[J - 1:J] + bb3, bf(0.0)),
                                -jnp.maximum(a3 + bb3, bf(0.0))], axis=-1)
        for i in range(0, J - 1, 2):
            a2 = jnp.concatenate([a3[i:i + 1], a3[i + 1:i + 2]], axis=-1)
            acc2 = acc2 + jnp.maximum(a2 + bb2, bf(0.0))
        nf2 = hc(jnp.maximum(
            abn[:, 2 * H:]
            + jnp.dot(acc2.reshape(R, 2 * H), w_en,
                      preferred_element_type=_F32)
            + b_n1e, 0.0))
        gi = jnp.dot(nf2, w_ih, preferred_element_type=_F32)
        gh = jnp.dot(hc(h), w_hh, preferred_element_type=_F32)
        rz = jax.nn.sigmoid(gi[:, :2 * H] + gh[:, :2 * H] + b_rz)
        r, z = rz[:, :H], rz[:, H:]
        g = jnp.tanh(gi[:, 2 * H:] + b_ig + r * (gh[:, 2 * H:] + b_hg))
        return g + z * (h - g)

    def enc_body(t, h):
        x64 = enc_ref[pl.ds(t, 1)].reshape(Bb, H)
        return step(embed_nf(x64), h)

    h = jax.lax.fori_loop(0, enc_ref.shape[0], enc_body,
                          jnp.zeros((R, H), _F32), unroll=2)

    pre0 = embed_pre(dec064_ref[...])
    pred0 = dec0r_ref[...].reshape(R, _IN)

    def dec_body(t, carry):
        pre, pred, h = carry
        h2 = step(hc(jnp.maximum(pre, 0.0)), h)
        delta = jnp.dot(h2.astype(bf), w_odh,
                        preferred_element_type=_F32) + b_od
        pred2 = pred + delta
        pred63 = jnp.concatenate(
            [pred2[j * Bb:(j + 1) * Bb] for j in range(J)], axis=1)
        out_ref[pl.ds(t, 1)] = pred63.reshape(1, Bb, J * _IN)
        pre2 = pre + jnp.dot(delta, w_in, preferred_element_type=_F32)
        return (pre2, pred2, h2)

    jax.lax.fori_loop(0, out_ref.shape[0], dec_body, (pre0, pred0, h),
                      unroll=2)


def kernel(encoder_input, decoder_input, W_in, b_in, W_e1, b_e1, W_e2, b_e2,
           W_n1, b_n1, W_i, b_i, W_h, b_h, W_out, b_out, W_dec, b_dec,
           send_idx, rec_idx):
    del send_idx, rec_idx
    T_src, B = encoder_input.shape[0], encoder_input.shape[1]
    T_tgt = decoder_input.shape[0]
    J, H = _J, _H
    F = J * _IN

    ones_col = lambda x: jnp.concatenate(
        [x, jnp.ones(x.shape[:-1] + (1,), _F32)], axis=-1)
    enc = ones_col(encoder_input.reshape(T_src, B, F))
    dec064 = ones_col(decoder_input[0].reshape(B, F))
    dec0r = decoder_input[0].transpose(1, 0, 2)
    W64 = jnp.concatenate(
        [jnp.kron(jnp.eye(J, dtype=_F32), W_in),
         jnp.tile(b_in, J).reshape(1, J * H)], axis=0)
    row = lambda v: v.reshape(1, -1)

    Bb = 256
    grid = (B // Bb,)

    wspec = lambda a: pl.BlockSpec(a.shape, lambda i: (0,) * a.ndim)
    weights = (W64, W_e1, row(b_e1), W_e2, row(b_e2),
               W_n1, row(b_n1), W_i, row(b_i), W_h, row(b_h),
               W_out, row(b_out), W_dec, row(b_dec), W_in)

    out = pl.pallas_call(
        _rgnn_kernel,
        grid=grid,
        in_specs=[
            pl.BlockSpec((T_src, Bb, H), lambda i: (0, i, 0)),
            pl.BlockSpec((Bb, H), lambda i: (i, 0)),
            pl.BlockSpec((J, Bb, _IN), lambda i: (0, i, 0)),
        ] + [wspec(w) for w in weights],
        out_specs=pl.BlockSpec((T_tgt, Bb, F), lambda i: (0, i, 0)),
        out_shape=jax.ShapeDtypeStruct((T_tgt, B, F), _F32),
        compiler_params=pltpu.CompilerParams(
            dimension_semantics=("parallel",)),
    )(enc, dec064, dec0r, *weights)
    return out
